# Initial kernel scaffold; baseline (speedup 1.0000x reference)
import jax
import jax.numpy as jnp
from jax.experimental import pallas as pl


def kernel(student_logits, targ_class, teacher_logits, features):
    raise NotImplementedError("write your pallas kernel here")


# speedup vs baseline: 14.6442x; 14.6442x over previous
import jax
import jax.numpy as jnp
from jax.experimental import pallas as pl
from jax.experimental.pallas import tpu as pltpu

MR_LAMBDA = 100.0
LABEL_SMOOTHING = 0.1
ECR_WARMUP_ITERATIONS = 1000.0
ITERATIONS = 0.0
KNN = 10
B, C, D = 512, 256, 128

_NEG_BIG = -3.0e38


def _spmt_body(sl_ref, tc_ref, tl_ref, f_ref, sup_ref, cons_ref):
    sl = sl_ref[:, :]

    m = jnp.max(sl, axis=1, keepdims=True)
    sh = sl - m
    es = jnp.exp(sh)
    se = jnp.sum(es, axis=1, keepdims=True)
    logp = sh - jnp.log(se)
    cols_c = jax.lax.broadcasted_iota(jnp.int32, (B, C), 1)
    onehot = cols_c == tc_ref[:, :]
    nll = -jnp.sum(jnp.where(onehot, logp, 0.0), axis=1)
    smooth = -jnp.sum(logp, axis=1) * (1.0 / C)
    per_ex = (1.0 - LABEL_SMOOTHING) * nll + LABEL_SMOOTHING * smooth
    sup_ref[:, :] = (jnp.sum(per_ex) * (1.0 / B)).reshape(1, 1)

    f = f_ref[:, :]
    gram = jnp.dot(f, f.T, preferred_element_type=jnp.float32)
    rn = jnp.sum(f * f, axis=1)
    sq = rn[:, None] + rn[None, :] - 2.0 * gram
    dist = jnp.sqrt(jnp.maximum(sq, 0.0))
    sims = 1.0 / (1.0 + dist)

    ps = es / se
    tl = tl_ref[:, :]
    mt = jnp.max(tl, axis=1, keepdims=True)
    et = jnp.exp(tl - mt)
    pt = et / jnp.sum(et, axis=1, keepdims=True)
    cross = jnp.dot(ps, pt.T, preferred_element_type=jnp.float32)
    pns = jnp.sum(ps * ps, axis=1)
    pnt = jnp.sum(pt * pt, axis=1)
    mse = (pns[:, None] + pnt[None, :] - 2.0 * cross) * (1.0 / C)

    prod = sims * mse
    cols_b = jax.lax.broadcasted_iota(jnp.int32, (B, B), 1)
    cur = sims
    acc = jnp.float32(0.0)
    for _ in range(KNN):
        rmax = jnp.max(cur, axis=1, keepdims=True)
        idx = jnp.min(jnp.where(cur == rmax, cols_b, jnp.int32(2**30)),
                      axis=1, keepdims=True)
        sel = cols_b == idx
        acc += jnp.sum(jnp.where(sel, prod, 0.0))
        cur = jnp.where(sel, _NEG_BIG, cur)
    cons = acc * (1.0 / (B * KNN))
    rampup = min(1.0, ITERATIONS / ECR_WARMUP_ITERATIONS)
    cons_ref[:, :] = ((MR_LAMBDA * rampup) * cons).reshape(1, 1)


def kernel(student_logits, targ_class, teacher_logits, features):
    targ2d = targ_class.reshape(B, 1)
    sup, cons = pl.pallas_call(
        _spmt_body,
        out_shape=(
            jax.ShapeDtypeStruct((1, 1), jnp.float32),
            jax.ShapeDtypeStruct((1, 1), jnp.float32),
        ),
    )(student_logits, targ2d, teacher_logits, features)
    supervised_loss = sup.reshape(1)
    cons_loss = cons.reshape(1)
    pseudo_loss = jnp.zeros((1,), jnp.float32)
    return (supervised_loss, cons_loss, pseudo_loss)


# speedup vs baseline: 22.5123x; 1.5373x over previous
import jax
import jax.numpy as jnp
from jax.experimental import pallas as pl
from jax.experimental.pallas import tpu as pltpu

MR_LAMBDA = 100.0
LABEL_SMOOTHING = 0.1
ECR_WARMUP_ITERATIONS = 1000.0
ITERATIONS = 0.0
KNN = 10
B, C, D = 512, 256, 128

_NEG_BIG = -3.0e38


def _spmt_body(sl_ref, tc_ref, tl_ref, f_ref, sup_ref, cons_ref):
    sl = sl_ref[:, :]

    m = jnp.max(sl, axis=1, keepdims=True)
    sh = sl - m
    es = jnp.exp(sh)
    se = jnp.sum(es, axis=1, keepdims=True)
    logp = sh - jnp.log(se)
    cols_c = jax.lax.broadcasted_iota(jnp.int32, (B, C), 1)
    onehot = cols_c == tc_ref[:, :]
    nll = -jnp.sum(jnp.where(onehot, logp, 0.0), axis=1)
    smooth = -jnp.sum(logp, axis=1) * (1.0 / C)
    per_ex = (1.0 - LABEL_SMOOTHING) * nll + LABEL_SMOOTHING * smooth
    sup_ref[:, :] = (jnp.sum(per_ex) * (1.0 / B)).reshape(1, 1)

    f = f_ref[:, :]
    gram = jnp.dot(f, f.T, preferred_element_type=jnp.float32)
    rn = jnp.sum(f * f, axis=1)
    sq = rn[:, None] + rn[None, :] - 2.0 * gram
    dist = jnp.sqrt(jnp.maximum(sq, 0.0))
    sims = 1.0 / (1.0 + dist)

    ps = es * (1.0 / se)
    tl = tl_ref[:, :]
    mt = jnp.max(tl, axis=1, keepdims=True)
    et = jnp.exp(tl - mt)
    pt = et * (1.0 / jnp.sum(et, axis=1, keepdims=True))
    cross = jnp.dot(ps, pt.T, preferred_element_type=jnp.float32)
    pns = jnp.sum(ps * ps, axis=1)
    pnt = jnp.sum(pt * pt, axis=1)
    mse = (pns[:, None] + pnt[None, :] - 2.0 * cross) * (1.0 / C)

    prod = sims * mse
    rows_i = jax.lax.broadcasted_iota(jnp.int32, (B, B), 0)
    cols_j = jax.lax.broadcasted_iota(jnp.int32, (B, B), 1)
    cur = jnp.where(rows_i == cols_j, _NEG_BIG, sims)
    for _ in range(KNN - 1):
        rmax = jnp.max(cur, axis=1, keepdims=True)
        cur = jnp.where(cur >= rmax, _NEG_BIG, cur)
    acc = jnp.sum(jnp.where(cur == _NEG_BIG, prod, 0.0))
    cons = acc * (1.0 / (B * KNN))
    rampup = min(1.0, ITERATIONS / ECR_WARMUP_ITERATIONS)
    cons_ref[:, :] = ((MR_LAMBDA * rampup) * cons).reshape(1, 1)


def kernel(student_logits, targ_class, teacher_logits, features):
    targ2d = targ_class.reshape(B, 1)
    sup, cons = pl.pallas_call(
        _spmt_body,
        out_shape=(
            jax.ShapeDtypeStruct((1, 1), jnp.float32),
            jax.ShapeDtypeStruct((1, 1), jnp.float32),
        ),
    )(student_logits, targ2d, teacher_logits, features)
    supervised_loss = sup.reshape(1)
    cons_loss = cons.reshape(1)
    pseudo_loss = jnp.zeros((1,), jnp.float32)
    return (supervised_loss, cons_loss, pseudo_loss)


# speedup vs baseline: 26.1367x; 1.1610x over previous
import jax
import jax.numpy as jnp
from jax.experimental import pallas as pl
from jax.experimental.pallas import tpu as pltpu

MR_LAMBDA = 100.0
LABEL_SMOOTHING = 0.1
ECR_WARMUP_ITERATIONS = 1000.0
ITERATIONS = 0.0
KNN = 10
B, C, D = 512, 256, 128

_NEG_BIG = -3.0e38


def _spmt_body(sl_ref, tc_ref, tl_ref, f_ref, sup_ref, cons_ref, pseudo_ref):
    sl = sl_ref[:, :]

    m = jnp.max(sl, axis=1, keepdims=True)
    sh = sl - m
    es = jnp.exp(sh)
    se = jnp.sum(es, axis=1, keepdims=True)
    logp = sh - jnp.log(se)
    cols_c = jax.lax.broadcasted_iota(jnp.int32, (B, C), 1)
    onehot = cols_c == tc_ref[:, :]
    nll = -jnp.sum(jnp.where(onehot, logp, 0.0), axis=1)
    smooth = -jnp.sum(logp, axis=1) * (1.0 / C)
    per_ex = (1.0 - LABEL_SMOOTHING) * nll + LABEL_SMOOTHING * smooth
    sup_ref[:] = (jnp.sum(per_ex) * (1.0 / B)).reshape(1)
    pseudo_ref[:] = jnp.zeros((1,), jnp.float32)

    f = f_ref[:, :]
    gram = jnp.dot(f, f.T, preferred_element_type=jnp.float32)
    rn = jnp.sum(f * f, axis=1)
    sq = rn[:, None] + rn[None, :] - 2.0 * gram
    dist = jnp.sqrt(jnp.maximum(sq, 0.0))
    sims = 1.0 / (1.0 + dist)

    ps = es * (1.0 / se)
    tl = tl_ref[:, :]
    mt = jnp.max(tl, axis=1, keepdims=True)
    et = jnp.exp(tl - mt)
    pt = et * (1.0 / jnp.sum(et, axis=1, keepdims=True))
    cross = jnp.dot(ps, pt.T, preferred_element_type=jnp.float32)
    pns = jnp.sum(ps * ps, axis=1)
    pnt = jnp.sum(pt * pt, axis=1)
    mse = (pns[:, None] + pnt[None, :] - 2.0 * cross) * (1.0 / C)

    prod = sims * mse
    rows_i = jax.lax.broadcasted_iota(jnp.int32, (B, B), 0)
    cols_j = jax.lax.broadcasted_iota(jnp.int32, (B, B), 1)
    cur = jnp.where(rows_i == cols_j, _NEG_BIG, sims)
    for _ in range(KNN - 1):
        rmax = jnp.max(cur, axis=1, keepdims=True)
        cur = jnp.where(cur >= rmax, _NEG_BIG, cur)
    acc = jnp.sum(jnp.where(cur == _NEG_BIG, prod, 0.0))
    cons = acc * (1.0 / (B * KNN))
    rampup = min(1.0, ITERATIONS / ECR_WARMUP_ITERATIONS)
    cons_ref[:] = ((MR_LAMBDA * rampup) * cons).reshape(1)


def kernel(student_logits, targ_class, teacher_logits, features):
    targ2d = targ_class.reshape(B, 1)
    sup, cons, pseudo = pl.pallas_call(
        _spmt_body,
        out_shape=(
            jax.ShapeDtypeStruct((1,), jnp.float32),
            jax.ShapeDtypeStruct((1,), jnp.float32),
            jax.ShapeDtypeStruct((1,), jnp.float32),
        ),
    )(student_logits, targ2d, teacher_logits, features)
    return (sup, cons, pseudo)
